# baseline (speedup 1.0000x reference)
import functools

import jax
import jax.numpy as jnp
from jax import lax
from jax.experimental import pallas as pl
from jax.experimental.pallas import tpu as pltpu
from jax.experimental.pallas import tpu_sc as plsc

T, H, F_D, F_M, E, K = 2048, 2048, 4096, 4096, 8, 2
EPS = 1e-6

BT = 128
NP = T * K + E * BT
NT = NP // BT
BF = 1024
BH = 1024

_INTERPRET = False


def _rms_rows(x):
    return x * lax.rsqrt(jnp.mean(x * x, axis=-1, keepdims=True) + EPS)


def _pre_body(x_ref, wrp_ref, rs_ref, wr_ref, pes_ref,
              xr_ref, i1_ref, i2_ref, w1_ref, w2_ref):
    xb = x_ref[...]
    xr = _rms_rows(xb) * (1.0 + wrp_ref[...])
    xr_ref[...] = xr
    yn = _rms_rows(xr) * rs_ref[...] * (H ** -0.5)
    logits = jnp.dot(yn, wr_ref[...], preferred_element_type=jnp.float32)
    rows = logits.shape[0]
    eio = lax.broadcasted_iota(jnp.int32, (rows, E), 1)
    m1 = jnp.max(logits, axis=1, keepdims=True)
    i1 = jnp.min(jnp.where(logits == m1, eio, E), axis=1, keepdims=True)
    l2 = jnp.where(eio == i1, -jnp.inf, logits)
    m2 = jnp.max(l2, axis=1, keepdims=True)
    i2 = jnp.min(jnp.where(l2 == m2, eio, E), axis=1, keepdims=True)
    r = jnp.exp(m2 - m1)
    w1 = 1.0 / (1.0 + r)
    w2 = r / (1.0 + r)
    pes = pes_ref[...]
    s1 = jnp.sum(jnp.where(eio == i1, pes, 0.0), axis=1, keepdims=True)
    s2 = jnp.sum(jnp.where(eio == i2, pes, 0.0), axis=1, keepdims=True)
    i1_ref[...] = i1
    i2_ref[...] = i2
    w1_ref[...] = w1 * s1
    w2_ref[...] = w2 * s2


def _run_pre(x, w_routed_pre, router_scale, Wr, per_expert_scale):
    bt = 256
    grid = (T // bt,)
    f32 = jnp.float32
    return pl.pallas_call(
        _pre_body,
        grid=grid,
        in_specs=[
            pl.BlockSpec((bt, H), lambda i: (i, 0)),
            pl.BlockSpec((1, H), lambda i: (0, 0)),
            pl.BlockSpec((1, H), lambda i: (0, 0)),
            pl.BlockSpec((H, E), lambda i: (0, 0)),
            pl.BlockSpec((1, E), lambda i: (0, 0)),
        ],
        out_specs=[
            pl.BlockSpec((bt, H), lambda i: (i, 0)),
            pl.BlockSpec((bt, 1), lambda i: (i, 0)),
            pl.BlockSpec((bt, 1), lambda i: (i, 0)),
            pl.BlockSpec((bt, 1), lambda i: (i, 0)),
            pl.BlockSpec((bt, 1), lambda i: (i, 0)),
        ],
        out_shape=[
            jax.ShapeDtypeStruct((T, H), f32),
            jax.ShapeDtypeStruct((T, 1), jnp.int32),
            jax.ShapeDtypeStruct((T, 1), jnp.int32),
            jax.ShapeDtypeStruct((T, 1), f32),
            jax.ShapeDtypeStruct((T, 1), f32),
        ],
        interpret=_INTERPRET,
    )(x, w_routed_pre.reshape(1, H), router_scale.reshape(1, H), Wr,
      per_expert_scale.reshape(1, E))


def _dispatch_metadata(i1, i2):
    i32 = jnp.int32
    e_pair = jnp.stack([i1[:, 0], i2[:, 0]], axis=1).reshape(-1)
    oh = (e_pair[:, None] == jnp.arange(E, dtype=i32)[None, :]).astype(i32)
    counts = jnp.sum(oh, axis=0)
    rank = jnp.sum((jnp.cumsum(oh, axis=0) - oh) * oh, axis=1)
    padded = ((counts + BT - 1) // BT) * BT
    offs = jnp.concatenate([jnp.zeros((1,), i32), jnp.cumsum(padded)])
    slot = offs[e_pair] + rank
    tok = jnp.arange(T * K, dtype=i32) // K
    src = jnp.zeros((NP,), i32).at[slot].set(tok)
    tile_starts = jnp.arange(NT, dtype=i32) * BT
    te = jnp.sum((tile_starts[:, None] >= offs[None, 1:]).astype(i32), axis=1)
    te = jnp.minimum(te, E - 1)
    return slot, src, te


def _sc_gather(table, idx):
    V, D = table.shape
    B = idx.shape[0]
    NW = 32
    b_per_w = B // NW
    CH = 16
    n_ch = b_per_w // CH
    mesh = plsc.VectorSubcoreMesh(core_axis_name="c", subcore_axis_name="s")

    @functools.partial(
        pl.kernel, mesh=mesh,
        out_type=jax.ShapeDtypeStruct((B, D), jnp.float32),
        scratch_types=[
            pltpu.VMEM((b_per_w,), jnp.int32),
            pltpu.VMEM((CH, D), jnp.float32),
            pltpu.VMEM((CH, D), jnp.float32),
            pltpu.SemaphoreType.DMA,
            pltpu.SemaphoreType.DMA,
            pltpu.SemaphoreType.DMA,
            pltpu.SemaphoreType.DMA,
        ],
    )
    def k(table_hbm, idx_hbm, out_hbm, idx_v, rows0, rows1, g0, g1, o0, o1):
        wid = lax.axis_index("s") * 2 + lax.axis_index("c")
        base = wid * b_per_w
        pltpu.sync_copy(idx_hbm.at[pl.ds(base, b_per_w)], idx_v)
        rows = (rows0, rows1)
        gsem = (g0, g1)
        osem = (o0, o1)
        gh = [None] * n_ch
        oh = [None] * n_ch
        for c in range(n_ch + 1):
            if c < n_ch:
                b = c % 2
                if c >= 2:
                    oh[c - 2].wait()
                gh[c] = pltpu.async_copy(
                    table_hbm.at[idx_v.at[pl.ds(c * CH, CH)]], rows[b], gsem[b])
            if c >= 1:
                p = (c - 1) % 2
                gh[c - 1].wait()
                oh[c - 1] = pltpu.async_copy(
                    rows[p], out_hbm.at[pl.ds(base + (c - 1) * CH, CH)], osem[p])
        oh[n_ch - 2].wait()
        oh[n_ch - 1].wait()

    return k(table, idx)


def _gate_up_body(te_ref, x_ref, wg_ref, wu_ref, o_ref):
    i = pl.program_id(1)
    xb = x_ref[pl.ds(i * BT, BT), :].astype(jnp.float32)
    g = jnp.dot(xb, wg_ref[0], preferred_element_type=jnp.float32)
    u = jnp.dot(xb, wu_ref[0], preferred_element_type=jnp.float32)
    o_ref[...] = (jax.nn.gelu(g, approximate=True) * u).astype(jnp.bfloat16)


def _run_expert_gate_up(X_s, Wg_e, Wu_e, te):
    grid = (F_M // BF, NT)
    spec = pltpu.PrefetchScalarGridSpec(
        num_scalar_prefetch=1,
        grid=grid,
        in_specs=[
            pl.BlockSpec((NP, H), lambda j, i, te: (0, 0)),
            pl.BlockSpec((1, H, BF), lambda j, i, te: (te[i], 0, j)),
            pl.BlockSpec((1, H, BF), lambda j, i, te: (te[i], 0, j)),
        ],
        out_specs=pl.BlockSpec((BT, BF), lambda j, i, te: (i, j)),
    )
    return pl.pallas_call(
        _gate_up_body,
        grid_spec=spec,
        out_shape=jax.ShapeDtypeStruct((NP, F_M), jnp.bfloat16),
        compiler_params=pltpu.CompilerParams(
            dimension_semantics=("arbitrary", "arbitrary")),
        interpret=_INTERPRET,
    )(te, X_s.astype(jnp.bfloat16), Wg_e, Wu_e)


def _down_body(te_ref, h_ref, wd_ref, o_ref):
    o_ref[...] = jnp.dot(h_ref[...].astype(jnp.float32), wd_ref[0],
                         preferred_element_type=jnp.float32)


def _run_expert_down(H_s, Wd_e, te):
    grid = (H // BH, NT)
    spec = pltpu.PrefetchScalarGridSpec(
        num_scalar_prefetch=1,
        grid=grid,
        in_specs=[
            pl.BlockSpec((BT, F_M), lambda n, i, te: (i, 0)),
            pl.BlockSpec((1, F_M, BH), lambda n, i, te: (te[i], 0, n)),
        ],
        out_specs=pl.BlockSpec((BT, BH), lambda n, i, te: (i, n)),
    )
    return pl.pallas_call(
        _down_body,
        grid_spec=spec,
        out_shape=jax.ShapeDtypeStruct((NP, H), jnp.float32),
        compiler_params=pltpu.CompilerParams(
            dimension_semantics=("arbitrary", "arbitrary")),
        interpret=_INTERPRET,
    )(te, H_s, Wd_e)


def _dense_gate_up_body(x_ref, wg_ref, wu_ref, o_ref):
    xb = x_ref[...]
    g = jnp.dot(xb, wg_ref[...], preferred_element_type=jnp.float32)
    u = jnp.dot(xb, wu_ref[...], preferred_element_type=jnp.float32)
    o_ref[...] = (jax.nn.gelu(g, approximate=True) * u).astype(jnp.bfloat16)


def _run_dense_gate_up(x, Wg, Wu):
    grid = (F_D // BF, T // BT)
    return pl.pallas_call(
        _dense_gate_up_body,
        grid=grid,
        in_specs=[
            pl.BlockSpec((BT, H), lambda j, i: (i, 0)),
            pl.BlockSpec((H, BF), lambda j, i: (0, j)),
            pl.BlockSpec((H, BF), lambda j, i: (0, j)),
        ],
        out_specs=pl.BlockSpec((BT, BF), lambda j, i: (i, j)),
        out_shape=jax.ShapeDtypeStruct((T, F_D), jnp.bfloat16),
        compiler_params=pltpu.CompilerParams(
            dimension_semantics=("arbitrary", "arbitrary")),
        interpret=_INTERPRET,
    )(x, Wg, Wu)


def _dense_down_body(h_ref, wd_ref, o_ref):
    o_ref[...] = jnp.dot(h_ref[...].astype(jnp.float32), wd_ref[...],
                         preferred_element_type=jnp.float32)


def _run_dense_down(Hd, Wd):
    grid = (H // BH, T // BT)
    return pl.pallas_call(
        _dense_down_body,
        grid=grid,
        in_specs=[
            pl.BlockSpec((BT, F_D), lambda n, i: (i, 0)),
            pl.BlockSpec((F_D, BH), lambda n, i: (0, n)),
        ],
        out_specs=pl.BlockSpec((BT, BH), lambda n, i: (i, n)),
        out_shape=jax.ShapeDtypeStruct((T, H), jnp.float32),
        compiler_params=pltpu.CompilerParams(
            dimension_semantics=("arbitrary", "arbitrary")),
        interpret=_INTERPRET,
    )(Hd, Wd)


def _final_body(z_ref, w1_ref, w2_ref, ydp_ref, wdp_ref, wrp2_ref, o_ref):
    zb = z_ref[...]
    a = zb[:, :H] * w1_ref[...] + zb[:, H:] * w2_ref[...]
    ym = _rms_rows(a) * (1.0 + wrp2_ref[...])
    yd = _rms_rows(ydp_ref[...]) * (1.0 + wdp_ref[...])
    o_ref[...] = yd + ym


def _run_final(Z, w1, w2, ydp, w_dense_post, w_routed_post):
    bt = 256
    grid = (T // bt,)
    return pl.pallas_call(
        _final_body,
        grid=grid,
        in_specs=[
            pl.BlockSpec((bt, K * H), lambda i: (i, 0)),
            pl.BlockSpec((bt, 1), lambda i: (i, 0)),
            pl.BlockSpec((bt, 1), lambda i: (i, 0)),
            pl.BlockSpec((bt, H), lambda i: (i, 0)),
            pl.BlockSpec((1, H), lambda i: (0, 0)),
            pl.BlockSpec((1, H), lambda i: (0, 0)),
        ],
        out_specs=pl.BlockSpec((bt, H), lambda i: (i, 0)),
        out_shape=jax.ShapeDtypeStruct((T, H), jnp.float32),
        interpret=_INTERPRET,
    )(Z.reshape(T, K * H), w1, w2, ydp,
      w_dense_post.reshape(1, H), w_routed_post.reshape(1, H))


def kernel(x, Wg_dense, Wu_dense, Wd_dense, w_dense_post, w_routed_pre,
           w_routed_post, router_scale, per_expert_scale, Wr, Wg_e, Wu_e, Wd_e):
    xr, i1, i2, w1, w2 = _run_pre(x, w_routed_pre, router_scale, Wr,
                                  per_expert_scale)
    slot, src, te = _dispatch_metadata(i1, i2)

    X_s = _sc_gather(xr, src)
    Hd = _run_dense_gate_up(x, Wg_dense, Wu_dense)
    H_s = _run_expert_gate_up(X_s, Wg_e, Wu_e, te)
    Y_s = _run_expert_down(H_s, Wd_e, te)
    Z = _sc_gather(Y_s, slot)
    ydp = _run_dense_down(Hd, Wd_dense)

    return _run_final(Z, w1, w2, ydp, w_dense_post, w_routed_post)


# speedup vs baseline: 1.0143x; 1.0143x over previous
import functools

import jax
import jax.numpy as jnp
from jax import lax
from jax.experimental import pallas as pl
from jax.experimental.pallas import tpu as pltpu
from jax.experimental.pallas import tpu_sc as plsc

T, H, F_D, F_M, E, K = 2048, 2048, 4096, 4096, 8, 2
EPS = 1e-6

BT = 128
NP = T * K + E * BT
NT = NP // BT
BF = 1024
BH = 1024


def _rms_rows(x):
    return x * lax.rsqrt(jnp.mean(x * x, axis=-1, keepdims=True) + EPS)


def _pre_body(x_ref, wrp_ref, rs_ref, wr_ref, pes_ref,
              xr_ref, i1_ref, i2_ref, w1_ref, w2_ref):
    xb = x_ref[...]
    xr = _rms_rows(xb) * (1.0 + wrp_ref[...])
    xr_ref[...] = xr
    yn = _rms_rows(xr) * rs_ref[...] * (H ** -0.5)
    logits = jnp.dot(yn, wr_ref[...], preferred_element_type=jnp.float32)
    rows = logits.shape[0]
    eio = lax.broadcasted_iota(jnp.int32, (rows, E), 1)
    m1 = jnp.max(logits, axis=1, keepdims=True)
    i1 = jnp.min(jnp.where(logits == m1, eio, E), axis=1, keepdims=True)
    l2 = jnp.where(eio == i1, -jnp.inf, logits)
    m2 = jnp.max(l2, axis=1, keepdims=True)
    i2 = jnp.min(jnp.where(l2 == m2, eio, E), axis=1, keepdims=True)
    r = jnp.exp(m2 - m1)
    w1 = 1.0 / (1.0 + r)
    w2 = r / (1.0 + r)
    pes = pes_ref[...]
    s1 = jnp.sum(jnp.where(eio == i1, pes, 0.0), axis=1, keepdims=True)
    s2 = jnp.sum(jnp.where(eio == i2, pes, 0.0), axis=1, keepdims=True)
    i1_ref[...] = i1
    i2_ref[...] = i2
    w1_ref[...] = w1 * s1
    w2_ref[...] = w2 * s2


def _run_pre(x, w_routed_pre, router_scale, Wr, per_expert_scale):
    bt = 256
    grid = (T // bt,)
    f32 = jnp.float32
    return pl.pallas_call(
        _pre_body,
        grid=grid,
        in_specs=[
            pl.BlockSpec((bt, H), lambda i: (i, 0)),
            pl.BlockSpec((1, H), lambda i: (0, 0)),
            pl.BlockSpec((1, H), lambda i: (0, 0)),
            pl.BlockSpec((H, E), lambda i: (0, 0)),
            pl.BlockSpec((1, E), lambda i: (0, 0)),
        ],
        out_specs=[
            pl.BlockSpec((bt, H), lambda i: (i, 0)),
            pl.BlockSpec((bt, 1), lambda i: (i, 0)),
            pl.BlockSpec((bt, 1), lambda i: (i, 0)),
            pl.BlockSpec((bt, 1), lambda i: (i, 0)),
            pl.BlockSpec((bt, 1), lambda i: (i, 0)),
        ],
        out_shape=[
            jax.ShapeDtypeStruct((T, H), f32),
            jax.ShapeDtypeStruct((T, 1), jnp.int32),
            jax.ShapeDtypeStruct((T, 1), jnp.int32),
            jax.ShapeDtypeStruct((T, 1), f32),
            jax.ShapeDtypeStruct((T, 1), f32),
        ],
    )(x, w_routed_pre.reshape(1, H), router_scale.reshape(1, H), Wr,
      per_expert_scale.reshape(1, E))


def _dispatch_metadata(i1, i2):
    i32 = jnp.int32
    e_pair = jnp.stack([i1[:, 0], i2[:, 0]], axis=1).reshape(-1)
    oh = (e_pair[:, None] == jnp.arange(E, dtype=i32)[None, :]).astype(i32)
    counts = jnp.sum(oh, axis=0)
    rank = jnp.sum((jnp.cumsum(oh, axis=0) - oh) * oh, axis=1)
    padded = ((counts + BT - 1) // BT) * BT
    offs = jnp.concatenate([jnp.zeros((1,), i32), jnp.cumsum(padded)])
    slot = offs[e_pair] + rank
    tok = jnp.arange(T * K, dtype=i32) // K
    src = jnp.zeros((NP,), i32).at[slot].set(tok)
    tile_starts = jnp.arange(NT, dtype=i32) * BT
    te = jnp.sum((tile_starts[:, None] >= offs[None, 1:]).astype(i32), axis=1)
    te = jnp.minimum(te, E - 1)
    return slot, src, te


def _sc_gather(table, idx):
    V, D = table.shape
    B = idx.shape[0]
    NW = 32
    b_per_w = B // NW
    CH = 16
    n_ch = b_per_w // CH
    mesh = plsc.VectorSubcoreMesh(core_axis_name="c", subcore_axis_name="s")

    @functools.partial(
        pl.kernel, mesh=mesh,
        out_type=jax.ShapeDtypeStruct((B, D), jnp.float32),
        scratch_types=[
            pltpu.VMEM((b_per_w,), jnp.int32),
            pltpu.VMEM((CH, D), jnp.float32),
            pltpu.VMEM((CH, D), jnp.float32),
            pltpu.SemaphoreType.DMA,
            pltpu.SemaphoreType.DMA,
            pltpu.SemaphoreType.DMA,
            pltpu.SemaphoreType.DMA,
        ],
    )
    def k(table_hbm, idx_hbm, out_hbm, idx_v, rows0, rows1, g0, g1, o0, o1):
        wid = lax.axis_index("s") * 2 + lax.axis_index("c")
        base = wid * b_per_w
        pltpu.sync_copy(idx_hbm.at[pl.ds(base, b_per_w)], idx_v)
        rows = (rows0, rows1)
        gsem = (g0, g1)
        osem = (o0, o1)
        gh = [None] * n_ch
        oh = [None] * n_ch
        for c in range(n_ch + 1):
            if c < n_ch:
                b = c % 2
                if c >= 2:
                    oh[c - 2].wait()
                gh[c] = pltpu.async_copy(
                    table_hbm.at[idx_v.at[pl.ds(c * CH, CH)]], rows[b], gsem[b])
            if c >= 1:
                p = (c - 1) % 2
                gh[c - 1].wait()
                oh[c - 1] = pltpu.async_copy(
                    rows[p], out_hbm.at[pl.ds(base + (c - 1) * CH, CH)], osem[p])
        oh[n_ch - 2].wait()
        oh[n_ch - 1].wait()

    return k(table, idx)


def _gate_up_body(te_ref, x_ref, wg_ref, wu_ref, o_ref):
    xb = x_ref[...]
    g = jnp.dot(xb, wg_ref[0], preferred_element_type=jnp.float32)
    u = jnp.dot(xb, wu_ref[0], preferred_element_type=jnp.float32)
    o_ref[...] = (jax.nn.gelu(g, approximate=True) * u).astype(jnp.bfloat16)


def _run_expert_gate_up(X_s, Wg_e, Wu_e, te):
    grid = (F_M // BF, NT)
    spec = pltpu.PrefetchScalarGridSpec(
        num_scalar_prefetch=1,
        grid=grid,
        in_specs=[
            pl.BlockSpec((BT, H), lambda j, i, te: (i, 0)),
            pl.BlockSpec((1, H, BF), lambda j, i, te: (te[i], 0, j)),
            pl.BlockSpec((1, H, BF), lambda j, i, te: (te[i], 0, j)),
        ],
        out_specs=pl.BlockSpec((BT, BF), lambda j, i, te: (i, j)),
    )
    return pl.pallas_call(
        _gate_up_body,
        grid_spec=spec,
        out_shape=jax.ShapeDtypeStruct((NP, F_M), jnp.bfloat16),
        compiler_params=pltpu.CompilerParams(
            dimension_semantics=("arbitrary", "arbitrary")),
    )(te, X_s, Wg_e, Wu_e)


def _down_body(te_ref, h_ref, wd_ref, o_ref):
    o_ref[...] = jnp.dot(h_ref[...].astype(jnp.float32), wd_ref[0],
                         preferred_element_type=jnp.float32)


def _run_expert_down(H_s, Wd_e, te):
    grid = (H // BH, NT)
    spec = pltpu.PrefetchScalarGridSpec(
        num_scalar_prefetch=1,
        grid=grid,
        in_specs=[
            pl.BlockSpec((BT, F_M), lambda n, i, te: (i, 0)),
            pl.BlockSpec((1, F_M, BH), lambda n, i, te: (te[i], 0, n)),
        ],
        out_specs=pl.BlockSpec((BT, BH), lambda n, i, te: (i, n)),
    )
    return pl.pallas_call(
        _down_body,
        grid_spec=spec,
        out_shape=jax.ShapeDtypeStruct((NP, H), jnp.float32),
        compiler_params=pltpu.CompilerParams(
            dimension_semantics=("arbitrary", "arbitrary")),
    )(te, H_s, Wd_e)


def _dense_gate_up_body(x_ref, wg_ref, wu_ref, o_ref):
    xb = x_ref[...]
    g = jnp.dot(xb, wg_ref[...], preferred_element_type=jnp.float32)
    u = jnp.dot(xb, wu_ref[...], preferred_element_type=jnp.float32)
    o_ref[...] = (jax.nn.gelu(g, approximate=True) * u).astype(jnp.bfloat16)


def _run_dense_gate_up(x, Wg, Wu):
    grid = (F_D // BF, T // BT)
    return pl.pallas_call(
        _dense_gate_up_body,
        grid=grid,
        in_specs=[
            pl.BlockSpec((BT, H), lambda j, i: (i, 0)),
            pl.BlockSpec((H, BF), lambda j, i: (0, j)),
            pl.BlockSpec((H, BF), lambda j, i: (0, j)),
        ],
        out_specs=pl.BlockSpec((BT, BF), lambda j, i: (i, j)),
        out_shape=jax.ShapeDtypeStruct((T, F_D), jnp.bfloat16),
        compiler_params=pltpu.CompilerParams(
            dimension_semantics=("arbitrary", "arbitrary")),
    )(x, Wg, Wu)


def _dense_down_body(h_ref, wd_ref, o_ref):
    o_ref[...] = jnp.dot(h_ref[...].astype(jnp.float32), wd_ref[...],
                         preferred_element_type=jnp.float32)


def _run_dense_down(Hd, Wd):
    grid = (H // BH, T // BT)
    return pl.pallas_call(
        _dense_down_body,
        grid=grid,
        in_specs=[
            pl.BlockSpec((BT, F_D), lambda n, i: (i, 0)),
            pl.BlockSpec((F_D, BH), lambda n, i: (0, n)),
        ],
        out_specs=pl.BlockSpec((BT, BH), lambda n, i: (i, n)),
        out_shape=jax.ShapeDtypeStruct((T, H), jnp.float32),
        compiler_params=pltpu.CompilerParams(
            dimension_semantics=("arbitrary", "arbitrary")),
    )(Hd, Wd)


def _final_body(z_ref, w1_ref, w2_ref, ydp_ref, wdp_ref, wrp2_ref, o_ref):
    zb = z_ref[...]
    a = zb[:, :H] * w1_ref[...] + zb[:, H:] * w2_ref[...]
    ym = _rms_rows(a) * (1.0 + wrp2_ref[...])
    yd = _rms_rows(ydp_ref[...]) * (1.0 + wdp_ref[...])
    o_ref[...] = yd + ym


def _run_final(Z, w1, w2, ydp, w_dense_post, w_routed_post):
    bt = 256
    grid = (T // bt,)
    return pl.pallas_call(
        _final_body,
        grid=grid,
        in_specs=[
            pl.BlockSpec((bt, K * H), lambda i: (i, 0)),
            pl.BlockSpec((bt, 1), lambda i: (i, 0)),
            pl.BlockSpec((bt, 1), lambda i: (i, 0)),
            pl.BlockSpec((bt, H), lambda i: (i, 0)),
            pl.BlockSpec((1, H), lambda i: (0, 0)),
            pl.BlockSpec((1, H), lambda i: (0, 0)),
        ],
        out_specs=pl.BlockSpec((bt, H), lambda i: (i, 0)),
        out_shape=jax.ShapeDtypeStruct((T, H), jnp.float32),
    )(Z.reshape(T, K * H), w1, w2, ydp,
      w_dense_post.reshape(1, H), w_routed_post.reshape(1, H))


def kernel(x, Wg_dense, Wu_dense, Wd_dense, w_dense_post, w_routed_pre,
           w_routed_post, router_scale, per_expert_scale, Wr, Wg_e, Wu_e, Wd_e):
    xr, i1, i2, w1, w2 = _run_pre(x, w_routed_pre, router_scale, Wr,
                                  per_expert_scale)
    slot, src, te = _dispatch_metadata(i1, i2)

    X_s = _sc_gather(xr, src)
    Hd = _run_dense_gate_up(x, Wg_dense, Wu_dense)
    H_s = _run_expert_gate_up(X_s, Wg_e, Wu_e, te)
    Y_s = _run_expert_down(H_s, Wd_e, te)
    Z = _sc_gather(Y_s, slot)
    ydp = _run_dense_down(Hd, Wd_dense)

    return _run_final(Z, w1, w2, ydp, w_dense_post, w_routed_post)
